# baseline (speedup 1.0000x reference)
import jax
import jax.numpy as jnp
from jax.experimental import pallas as pl
from jax.experimental.pallas import tpu as pltpu

_TEMP = 0.1
_NSAMP = 10
_H = 224
_W = 224
_HW = _H * _W
_LANES = 128
_CHUNKS = _HW // _LANES


def _csum_rows(v, n):
    sh = 1
    while sh < n:
        v = v + jnp.pad(v, ((sh, 0), (0, 0)))[:n]
        sh *= 2
    return v


def _csum_lanes(v, n):
    sh = 1
    while sh < n:
        v = v + jnp.pad(v, ((0, 0), (sh, 0)))[:, :n]
        sh *= 2
    return v


def _row_body(u_ref, x_ref, o_ref, e_ref):
    x = x_ref[0]
    lg = x / jnp.float32(_TEMP)
    m = jnp.max(lg)
    e = jnp.exp(lg - m)
    e_ref[...] = e
    s = jnp.sum(e, axis=1, keepdims=True)
    p = _csum_rows(s, _CHUNKS)
    z = p[_CHUNKS - 1, 0]
    acc_x = jnp.float32(0.0)
    acc_y = jnp.float32(0.0)
    for j in range(_NSAMP):
        t = u_ref[0, 0, j] * z
        mask = p < t
        nf = jnp.sum(mask.astype(jnp.int32))
        pm = jnp.max(jnp.where(mask, p, jnp.float32(0.0)))
        chunk = e_ref[pl.ds(nf, 1), :]
        c = _csum_lanes(chunk, _LANES)
        w = jnp.sum((c < (t - pm)).astype(jnp.int32))
        idx = jnp.minimum(nf * _LANES + w, _HW - 1)
        acc_x += (idx % _W).astype(jnp.float32)
        acc_y += (idx // _W).astype(jnp.float32)
    px = (acc_x / jnp.float32(_W * _NSAMP)).reshape(1, 1)
    py = (acc_y / jnp.float32(_H * _NSAMP)).reshape(1, 1)
    o_ref[0] = jnp.concatenate([px, py], axis=1)


def kernel(heatmap):
    B, C, H, W = heatmap.shape
    n = B * C
    hm = heatmap.reshape(n, _CHUNKS, _LANES)
    skey = jax.random.fold_in(jax.random.key(0), 1)
    u = jax.random.uniform(skey, (n, _NSAMP), dtype=heatmap.dtype)
    u3 = u.reshape(n, 1, _NSAMP)
    out = pl.pallas_call(
        _row_body,
        grid=(n,),
        in_specs=[
            pl.BlockSpec((1, 1, _NSAMP), lambda i: (i, 0, 0)),
            pl.BlockSpec((1, _CHUNKS, _LANES), lambda i: (i, 0, 0)),
        ],
        out_specs=pl.BlockSpec((1, 1, 2), lambda i: (i, 0, 0)),
        out_shape=jax.ShapeDtypeStruct((n, 1, 2), jnp.float32),
        scratch_shapes=[pltpu.VMEM((_CHUNKS, _LANES), jnp.float32)],
    )(u3, hm)
    return out.reshape(B, C, 2)


# speedup vs baseline: 5.1496x; 5.1496x over previous
import functools

import jax
import jax.numpy as jnp
from jax import lax
from jax.experimental import pallas as pl
from jax.experimental.pallas import tpu as pltpu
from jax.experimental.pallas import tpu_sc as plsc

_TEMP = 0.1
_NSAMP = 10
_H = 224
_W = 224
_HW = _H * _W
_LANES = 128
_CHUNKS = _HW // _LANES
_ROWS = 768
_RPB = 8
_REC = 512
_MLANE = _CHUNKS
_NW = 32
_RPW = _ROWS // _NW


def _csum_lanes(v, n):
    sh = 1
    while sh < n:
        v = v + jnp.pad(v, ((0, 0), (sh, 0)))[:, :n]
        sh *= 2
    return v


def _dense_body(x_ref, o_ref, f_ref):
    ones_row = jnp.ones((1, _LANES), jnp.float32)
    eye = (lax.broadcasted_iota(jnp.int32, (_CHUNKS, _CHUNKS), 0) ==
           lax.broadcasted_iota(jnp.int32, (_CHUNKS, _CHUNKS), 1)
           ).astype(jnp.float32)
    for r in range(_RPB):
        x = x_ref[r]
        lg = x / jnp.float32(_TEMP)
        m = jnp.max(lg)
        e = jnp.exp(lg - m)
        srow = lax.dot_general(ones_row, e, (((1,), (1,)), ((), ())),
                               precision=lax.Precision.HIGHEST,
                               preferred_element_type=jnp.float32)
        p = _csum_lanes(srow, _CHUNKS)
        mv = jnp.full((1, 8), m, jnp.float32)
        pad = jnp.zeros((1, _REC - _CHUNKS - 8), jnp.float32)
        o_ref[r] = jnp.concatenate([p, mv, pad], axis=1)
        pexc_row = jnp.concatenate(
            [jnp.zeros((1, 1), jnp.float32), p[:, :_CHUNKS - 1]], axis=1)
        pexc_col = lax.dot_general(eye, pexc_row, (((1,), (1,)), ((), ())),
                                   precision=lax.Precision.HIGHEST,
                                   preferred_element_type=jnp.float32)
        c_loc = _csum_lanes(e, _LANES)
        f_ref[r] = c_loc + pexc_col


def _splat(ref, lane):
    idx = jnp.full((16,), lane, jnp.int32)
    return plsc.load_gather(ref, [idx])


def _sc_body(pm_hbm, u_hbm, f_hbm, o_hbm,
             pm_v, u_v, idx_v, chunk_v, o_v, sem):
    cid = lax.axis_index("c")
    sid = lax.axis_index("s")
    wid = sid * 2 + cid

    def row_body(k, carry):
        row = wid * _RPW + k
        pltpu.sync_copy(pm_hbm.at[row], pm_v)
        pltpu.sync_copy(u_hbm.at[row], u_v)
        uu = u_v[...]
        z = _splat(pm_v, _CHUNKS - 1)
        t = uu * z
        lo = jnp.zeros((16,), jnp.int32)
        hi = jnp.full((16,), _CHUNKS, jnp.int32)
        for _ in range(9):
            mid = lax.shift_right_arithmetic(lo + hi, 1)
            pmid = plsc.load_gather(pm_v, [mid])
            cond = pmid < t
            lo = jnp.where(cond, mid + 1, lo)
            hi = jnp.where(cond, hi, mid)
        nf = lo
        idx_v[...] = row * _CHUNKS + nf
        pltpu.async_copy(f_hbm.at[idx_v], chunk_v, sem).wait()
        lane = lax.iota(jnp.int32, 16)
        lo2 = jnp.zeros((16,), jnp.int32)
        hi2 = jnp.full((16,), _LANES, jnp.int32)
        for _ in range(7):
            mid2 = lax.shift_right_arithmetic(lo2 + hi2, 1)
            fv = plsc.load_gather(chunk_v, [lane, mid2])
            cond2 = fv < t
            lo2 = jnp.where(cond2, mid2 + 1, lo2)
            hi2 = jnp.where(cond2, hi2, mid2)
        idx = jnp.minimum(nf * _LANES + lo2, _HW - 1)
        xq = (idx % _W).astype(jnp.float32)
        yq = (idx // _W).astype(jnp.float32)
        live = lane < _NSAMP
        px = jnp.sum(jnp.where(live, xq, 0.0)) * jnp.float32(1.0 / (_W * _NSAMP))
        py = jnp.sum(jnp.where(live, yq, 0.0)) * jnp.float32(1.0 / (_H * _NSAMP))
        o_v[...] = jnp.where(lane == 0, px, jnp.where(lane == 1, py, 0.0))
        pltpu.sync_copy(o_v, o_hbm.at[row])
        return carry

    lax.fori_loop(0, _RPW, row_body, jnp.int32(0))


_sc_sample_cache = []


def _get_sc_sample():
    if not _sc_sample_cache:
        _sc_sample_cache.append(functools.partial(
            pl.kernel,
            mesh=plsc.VectorSubcoreMesh(
                core_axis_name="c", subcore_axis_name="s"),
            compiler_params=pltpu.CompilerParams(needs_layout_passes=False),
            out_type=jax.ShapeDtypeStruct((_ROWS, 16), jnp.float32),
            scratch_types=[
                pltpu.VMEM((_REC,), jnp.float32),
                pltpu.VMEM((16,), jnp.float32),
                pltpu.VMEM((16,), jnp.int32),
                pltpu.VMEM((16, _LANES), jnp.float32),
                pltpu.VMEM((16,), jnp.float32),
                pltpu.SemaphoreType.DMA,
            ],
        )(_sc_body))
    return _sc_sample_cache[0]


def kernel(heatmap):
    B, C, H, W = heatmap.shape
    n = B * C
    hm3 = heatmap.reshape(n, _CHUNKS, _LANES)
    pm, f = pl.pallas_call(
        _dense_body,
        grid=(n // _RPB,),
        in_specs=[pl.BlockSpec((_RPB, _CHUNKS, _LANES), lambda i: (i, 0, 0))],
        out_specs=[
            pl.BlockSpec((_RPB, 1, _REC), lambda i: (i, 0, 0)),
            pl.BlockSpec((_RPB, _CHUNKS, _LANES), lambda i: (i, 0, 0)),
        ],
        out_shape=[
            jax.ShapeDtypeStruct((n, 1, _REC), jnp.float32),
            jax.ShapeDtypeStruct((n, _CHUNKS, _LANES), jnp.float32),
        ],
    )(hm3)
    pm2 = pm.reshape(n, _REC)
    skey = jax.random.fold_in(jax.random.key(0), 1)
    u = jax.random.uniform(skey, (n, _NSAMP), dtype=heatmap.dtype)
    u2 = jnp.concatenate(
        [u, jnp.full((n, 16 - _NSAMP), 0.5, heatmap.dtype)], axis=1)
    f2 = f.reshape(n * _CHUNKS, _LANES)
    o = _get_sc_sample()(pm2, u2, f2)
    return o[:, :2].reshape(B, C, 2)


# speedup vs baseline: 8.8081x; 1.7105x over previous
import functools

import jax
import jax.numpy as jnp
from jax import lax
from jax.experimental import pallas as pl
from jax.experimental.pallas import tpu as pltpu
from jax.experimental.pallas import tpu_sc as plsc

_TEMP = 0.1
_NSAMP = 10
_H = 224
_W = 224
_HW = _H * _W
_LANES = 128
_CHUNKS = _HW // _LANES
_ROWS = 768
_RPB = 8
_REC = 512
_MLANE = _CHUNKS
_NW = 32
_RPW = _ROWS // _NW


def _csum_lanes(v, n):
    sh = 1
    while sh < n:
        v = v + jnp.pad(v, ((0, 0), (sh, 0)))[:, :n]
        sh *= 2
    return v


def _dense_body(x_ref, o_ref, f_ref):
    ones_row = jnp.ones((1, _LANES), jnp.float32)
    r392 = lax.broadcasted_iota(jnp.int32, (_CHUNKS, _CHUNKS), 0)
    c392 = lax.broadcasted_iota(jnp.int32, (_CHUNKS, _CHUNKS), 1)
    tri392 = (r392 <= c392).astype(jnp.float32)
    lts392 = (c392 < r392).astype(jnp.float32)
    r128 = lax.broadcasted_iota(jnp.int32, (_LANES, _LANES), 0)
    c128 = lax.broadcasted_iota(jnp.int32, (_LANES, _LANES), 1)
    tri128 = (r128 <= c128).astype(jnp.float32)
    for r in range(_RPB):
        x = x_ref[r]
        lg = x / jnp.float32(_TEMP)
        m = jnp.max(lg)
        e = jnp.exp(lg - m)
        srow = lax.dot_general(ones_row, e, (((1,), (1,)), ((), ())),
                               precision=lax.Precision.HIGHEST,
                               preferred_element_type=jnp.float32)
        p = lax.dot_general(srow, tri392, (((1,), (0,)), ((), ())),
                            precision=lax.Precision.HIGHEST,
                            preferred_element_type=jnp.float32)
        mv = jnp.full((1, 8), m, jnp.float32)
        pad = jnp.zeros((1, _REC - _CHUNKS - 8), jnp.float32)
        o_ref[r] = jnp.concatenate([p, mv, pad], axis=1)
        pexc_col = lax.dot_general(lts392, srow, (((1,), (1,)), ((), ())),
                                   precision=lax.Precision.HIGHEST,
                                   preferred_element_type=jnp.float32)
        c_loc = lax.dot_general(e, tri128, (((1,), (0,)), ((), ())),
                                preferred_element_type=jnp.float32)
        f_ref[r] = c_loc + pexc_col


def _splat(ref, lane):
    idx = jnp.full((16,), lane, jnp.int32)
    return plsc.load_gather(ref, [idx])


_GB = 8


def _sc_body(pm_hbm, u_hbm, f_hbm, o_hbm,
             pm_v, u_v, idx_v, chunk_v, o_v, sem):
    cid = lax.axis_index("c")
    sid = lax.axis_index("s")
    wid = sid * 2 + cid
    base = wid * _RPW
    pltpu.sync_copy(pm_hbm.at[pl.ds(base, _RPW)], pm_v)
    pltpu.sync_copy(u_hbm.at[pl.ds(base, _RPW)], u_v)
    lane = lax.iota(jnp.int32, 16)
    live = lane < _NSAMP
    for b in range(_RPW // _GB):
        nfs = []
        for rr in range(_GB):
            k = b * _GB + rr
            uu = u_v[k, :]
            krow = (uu * 0.0).astype(jnp.int32) + k
            z = plsc.load_gather(pm_v, [krow, lane * 0 + (_CHUNKS - 1)])
            t = uu * z
            lo = jnp.zeros((16,), jnp.int32)
            hi = jnp.full((16,), _CHUNKS, jnp.int32)
            for _ in range(9):
                mid = lax.shift_right_arithmetic(lo + hi, 1)
                pmid = plsc.load_gather(pm_v, [krow, mid])
                cond = pmid < t
                lo = jnp.where(cond, mid + 1, lo)
                hi = jnp.where(cond, hi, mid)
            nfs.append((lo, t))
            idx_v[pl.ds(rr * 16, 16)] = (base + k) * _CHUNKS + lo
        pltpu.async_copy(f_hbm.at[idx_v], chunk_v, sem).wait()
        for rr in range(_GB):
            k = b * _GB + rr
            nf, t = nfs[rr]
            lo2 = jnp.zeros((16,), jnp.int32)
            hi2 = jnp.full((16,), _LANES, jnp.int32)
            for _ in range(7):
                mid2 = lax.shift_right_arithmetic(lo2 + hi2, 1)
                fv = plsc.load_gather(chunk_v, [rr * 16 + lane, mid2])
                cond2 = fv < t
                lo2 = jnp.where(cond2, mid2 + 1, lo2)
                hi2 = jnp.where(cond2, hi2, mid2)
            idx = jnp.minimum(nf * _LANES + lo2, _HW - 1)
            xq = (idx % _W).astype(jnp.float32)
            yq = (idx // _W).astype(jnp.float32)
            px = jnp.sum(jnp.where(live, xq, 0.0)) * jnp.float32(
                1.0 / (_W * _NSAMP))
            py = jnp.sum(jnp.where(live, yq, 0.0)) * jnp.float32(
                1.0 / (_H * _NSAMP))
            o_v[k, :] = jnp.where(lane == 0, px, jnp.where(lane == 1, py, 0.0))
    pltpu.sync_copy(o_v, o_hbm.at[pl.ds(base, _RPW)])


_sc_sample_cache = []


def _get_sc_sample():
    if not _sc_sample_cache:
        _sc_sample_cache.append(functools.partial(
            pl.kernel,
            mesh=plsc.VectorSubcoreMesh(
                core_axis_name="c", subcore_axis_name="s"),
            compiler_params=pltpu.CompilerParams(needs_layout_passes=False),
            out_type=jax.ShapeDtypeStruct((_ROWS, 16), jnp.float32),
            scratch_types=[
                pltpu.VMEM((_RPW, _REC), jnp.float32),
                pltpu.VMEM((_RPW, 16), jnp.float32),
                pltpu.VMEM((_GB * 16,), jnp.int32),
                pltpu.VMEM((_GB * 16, _LANES), jnp.float32),
                pltpu.VMEM((_RPW, 16), jnp.float32),
                pltpu.SemaphoreType.DMA,
            ],
        )(_sc_body))
    return _sc_sample_cache[0]


def kernel(heatmap):
    B, C, H, W = heatmap.shape
    n = B * C
    hm3 = heatmap.reshape(n, _CHUNKS, _LANES)
    pm, f = pl.pallas_call(
        _dense_body,
        grid=(n // _RPB,),
        in_specs=[pl.BlockSpec((_RPB, _CHUNKS, _LANES), lambda i: (i, 0, 0))],
        out_specs=[
            pl.BlockSpec((_RPB, 1, _REC), lambda i: (i, 0, 0)),
            pl.BlockSpec((_RPB, _CHUNKS, _LANES), lambda i: (i, 0, 0)),
        ],
        out_shape=[
            jax.ShapeDtypeStruct((n, 1, _REC), jnp.float32),
            jax.ShapeDtypeStruct((n, _CHUNKS, _LANES), jnp.float32),
        ],
    )(hm3)
    pm2 = pm.reshape(n, _REC)
    skey = jax.random.fold_in(jax.random.key(0), 1)
    u = jax.random.uniform(skey, (n, _NSAMP), dtype=heatmap.dtype)
    u2 = jnp.concatenate(
        [u, jnp.full((n, 16 - _NSAMP), 0.5, heatmap.dtype)], axis=1)
    f2 = f.reshape(n * _CHUNKS, _LANES)
    o = _get_sc_sample()(pm2, u2, f2)
    return o[:, :2].reshape(B, C, 2)
